# Initial kernel scaffold; baseline (speedup 1.0000x reference)
import jax
import jax.numpy as jnp
from jax.experimental import pallas as pl


def kernel(xyz, features, query_rv_xyz, query_rv_coords, rv_map):
    raise NotImplementedError("write your pallas kernel here")


# speedup vs baseline: 17.5574x; 17.5574x over previous
import functools

import jax
import jax.numpy as jnp
import numpy as np
from jax import lax
from jax.experimental import pallas as pl
from jax.experimental.pallas import tpu as pltpu
from jax.experimental.pallas import tpu_sc as plsc

_RADIUS2 = 9.0
_NSAMPLE = 16
_H_RANGE, _W_RANGE = 5, 11
_KPAD = 64


def _window_tables():
    k = np.arange(_KPAD)
    h = k // _W_RANGE - _H_RANGE // 2
    w = k % _W_RANGE - _W_RANGE // 2
    pad = k >= _H_RANGE * _W_RANGE
    hrel = np.where(pad, 10000, h).astype(np.int32)
    wrel = np.where(pad, 10000, w).astype(np.int32)
    return jnp.asarray(hrel), jnp.asarray(wrel)


def _sc_kernel(xh, yh, zh, features, rv_flat, rows, cols, qx, qy, qz,
               hrel, wrel):
    C = features.shape[1] // 2
    M = rows.shape[0]
    RH, RW = 64, 2048
    NW = 32
    QPT = M // NW
    CO = 3 + C
    ROW = CO * _NSAMPLE

    mesh = plsc.VectorSubcoreMesh(core_axis_name="c", subcore_axis_name="s")

    @functools.partial(
        pl.kernel,
        out_type=jax.ShapeDtypeStruct((M, ROW), jnp.float32),
        mesh=mesh,
        scratch_types=[
            pltpu.VMEM((QPT,), jnp.int32),
            pltpu.VMEM((QPT,), jnp.int32),
            pltpu.VMEM((QPT,), jnp.float32),
            pltpu.VMEM((QPT,), jnp.float32),
            pltpu.VMEM((QPT,), jnp.float32),
            pltpu.VMEM((_KPAD,), jnp.int32),
            pltpu.VMEM((_KPAD,), jnp.int32),
            pltpu.VMEM((_KPAD,), jnp.int32),
            pltpu.VMEM((_KPAD,), jnp.int32),
            pltpu.VMEM((_KPAD,), jnp.int32),
            pltpu.VMEM((_KPAD,), jnp.float32),
            pltpu.VMEM((_KPAD,), jnp.float32),
            pltpu.VMEM((_KPAD,), jnp.float32),
            pltpu.VMEM((80,), jnp.int32),
            pltpu.VMEM((80,), jnp.float32),
            pltpu.VMEM((80,), jnp.float32),
            pltpu.VMEM((80,), jnp.float32),
            pltpu.VMEM((_NSAMPLE,), jnp.int32),
            pltpu.VMEM((_NSAMPLE, 2 * C), jnp.float32),
            pltpu.VMEM((ROW,), jnp.float32),
            pltpu.SemaphoreType.DMA,
        ],
        compiler_params=pltpu.CompilerParams(needs_layout_passes=False),
    )
    def k(xh_h, yh_h, zh_h, feat_h, rv_h, rows_h, cols_h, qx_h, qy_h, qz_h,
          hrel_h, wrel_h, out_h, rows_v, cols_v, qx_v, qy_v, qz_v, hrel_v,
          wrel_v, idxb, okb, candb, cx, cy, cz, seli, selx, sely, selz,
          idxg, featb, obuf, sem):
        wid = lax.axis_index("s") * 2 + lax.axis_index("c")
        base = wid * QPT
        pltpu.sync_copy(rows_h.at[pl.ds(base, QPT)], rows_v)
        pltpu.sync_copy(cols_h.at[pl.ds(base, QPT)], cols_v)
        pltpu.sync_copy(qx_h.at[pl.ds(base, QPT)], qx_v)
        pltpu.sync_copy(qy_h.at[pl.ds(base, QPT)], qy_v)
        pltpu.sync_copy(qz_h.at[pl.ds(base, QPT)], qz_v)
        pltpu.sync_copy(hrel_h, hrel_v)
        pltpu.sync_copy(wrel_h, wrel_v)

        iota = lax.iota(jnp.int32, 16)
        iota16x = iota * _NSAMPLE

        def do_query(i, r, c, qxs, qys, qzs):
            for j in range(_KPAD // 16):
                hr = hrel_v[pl.ds(j * 16, 16)]
                wr = wrel_v[pl.ds(j * 16, 16)]
                rp = r + hr
                cp = c + wr
                ok = (rp >= 0) & (rp < RH) & (cp >= 0) & (cp < RW)
                rc = jnp.clip(rp, 0, RH - 1)
                cc = jnp.clip(cp, 0, RW - 1)
                idxb[pl.ds(j * 16, 16)] = rc * RW + cc
                okb[pl.ds(j * 16, 16)] = jnp.where(ok, 1, 0)
            pltpu.async_copy(rv_h.at[idxb], candb, sem).wait()
            pltpu.async_copy(xh_h.at[candb], cx, sem).wait()
            pltpu.async_copy(yh_h.at[candb], cy, sem).wait()
            pltpu.async_copy(zh_h.at[candb], cz, sem).wait()

            cnt = jnp.int32(0)
            for j in range(_KPAD // 16):
                cand = candb[pl.ds(j * 16, 16)]
                ok = okb[pl.ds(j * 16, 16)] != 0
                xs = cx[pl.ds(j * 16, 16)]
                ys = cy[pl.ds(j * 16, 16)]
                zs = cz[pl.ds(j * 16, 16)]
                dx = xs - qxs
                dy = ys - qys
                dz = zs - qzs
                d2 = dx * dx + dy * dy + dz * dz
                inb = ok & (cand >= 0) & (d2 <= _RADIUS2)
                bi = jnp.where(inb, 1, 0)
                pos = plsc.cumsum(bi) - 1 + cnt
                plsc.store_scatter(seli, [pos], cand, mask=inb)
                plsc.store_scatter(selx, [pos], xs, mask=inb)
                plsc.store_scatter(sely, [pos], ys, mask=inb)
                plsc.store_scatter(selz, [pos], zs, mask=inb)
                cnt = cnt + jnp.sum(bi)

            nonempty = cnt > 0
            pos16 = jnp.where(iota < cnt, iota, 0)
            gi = plsc.load_gather(seli, [pos16])
            gx = plsc.load_gather(selx, [pos16])
            gy = plsc.load_gather(sely, [pos16])
            gz = plsc.load_gather(selz, [pos16])
            gi = jnp.where(nonempty, gi, 0)
            idxg[...] = lax.shift_right_logical(gi, 1)
            par = gi & 1
            pltpu.async_copy(feat_h.at[idxg], featb, sem).wait()

            zf = jnp.where(nonempty, jnp.float32(1.0), jnp.float32(0.0))
            obuf[pl.ds(0, 16)] = jnp.where(nonempty, gx - qxs, 0.0)
            obuf[pl.ds(16, 16)] = jnp.where(nonempty, gy - qys, 0.0)
            obuf[pl.ds(32, 16)] = jnp.where(nonempty, gz - qzs, 0.0)
            for s in range(_NSAMPLE):
                off = pl.multiple_of(par[s] * C, C)
                for cb in range(C // 16):
                    vec = featb[s, pl.ds(off + cb * 16, 16)]
                    posv = iota16x + (48 + cb * 256 + s)
                    plsc.store_scatter(obuf, [posv], vec * zf)
            pltpu.sync_copy(obuf, out_h.at[base + i])

        def body(g, carry):
            start = pl.multiple_of(g * 16, 16)
            rows16 = rows_v[pl.ds(start, 16)]
            cols16 = cols_v[pl.ds(start, 16)]
            qx16 = qx_v[pl.ds(start, 16)]
            qy16 = qy_v[pl.ds(start, 16)]
            qz16 = qz_v[pl.ds(start, 16)]
            for qq in range(16):
                do_query(g * 16 + qq, rows16[qq], cols16[qq],
                         qx16[qq], qy16[qq], qz16[qq])
            return carry

        lax.fori_loop(0, QPT // 16, body, jnp.int32(0))

    return k(xh, yh, zh, features, rv_flat, rows, cols, qx, qy, qz,
             hrel, wrel)


def kernel(xyz, features, query_rv_xyz, query_rv_coords, rv_map):
    M = query_rv_xyz.shape[0]
    C = features.shape[1]
    features = features.reshape(features.shape[0] // 2, 2 * C)
    rv_flat = rv_map.reshape(-1)
    rows = query_rv_coords[:, 1].astype(jnp.int32)
    cols = query_rv_coords[:, 2].astype(jnp.int32)
    xh = xyz[:, 0]
    yh = xyz[:, 1]
    zh = xyz[:, 2]
    qx = query_rv_xyz[:, 0]
    qy = query_rv_xyz[:, 1]
    qz = query_rv_xyz[:, 2]
    hrel, wrel = _window_tables()
    out = _sc_kernel(xh, yh, zh, features, rv_flat, rows, cols, qx, qy, qz,
                     hrel, wrel)
    return out.reshape(M, 3 + C, _NSAMPLE)


# speedup vs baseline: 28.0863x; 1.5997x over previous
import functools

import jax
import jax.numpy as jnp
import numpy as np
from jax import lax
from jax.experimental import pallas as pl
from jax.experimental.pallas import tpu as pltpu
from jax.experimental.pallas import tpu_sc as plsc

_RADIUS2 = 9.0
_NSAMPLE = 16
_H_RANGE, _W_RANGE = 5, 11
_KPAD = 64
_RH, _RW = 64, 2048


def _sc_kernel(xh, yh, zh, features, rv_flat, rows, cols, qx, qy, qz):
    C = features.shape[1] // 2
    M = rows.shape[0]
    NW = 32
    QPT = M // NW
    CO = 3 + C
    ROW = CO * _NSAMPLE
    G = 16
    NG = QPT // G

    mesh = plsc.VectorSubcoreMesh(core_axis_name="c", subcore_axis_name="s")

    @functools.partial(
        pl.kernel,
        out_type=jax.ShapeDtypeStruct((M * ROW,), jnp.float32),
        mesh=mesh,
        scratch_types=[
            pltpu.VMEM((QPT,), jnp.int32),
            pltpu.VMEM((QPT,), jnp.int32),
            pltpu.VMEM((QPT,), jnp.float32),
            pltpu.VMEM((QPT,), jnp.float32),
            pltpu.VMEM((QPT,), jnp.float32),
            pltpu.VMEM((8, 128), jnp.int32),
            pltpu.VMEM((8, 128), jnp.int32),
            pltpu.VMEM((8, 128), jnp.float32),
            pltpu.VMEM((8, 128), jnp.float32),
            pltpu.VMEM((8, 128), jnp.float32),
            pltpu.VMEM((80,), jnp.int32),
            pltpu.VMEM((80,), jnp.float32),
            pltpu.VMEM((80,), jnp.float32),
            pltpu.VMEM((80,), jnp.float32),
            pltpu.VMEM((4, 64), jnp.int32),
            pltpu.VMEM((G, 16), jnp.int32),
            pltpu.VMEM((G, 16), jnp.float32),
            pltpu.VMEM((2, 64, 2 * C), jnp.float32),
            pltpu.VMEM((G * ROW,), jnp.float32),
            pltpu.SemaphoreType.DMA,
            pltpu.SemaphoreType.DMA,
            pltpu.SemaphoreType.DMA,
        ],
        compiler_params=pltpu.CompilerParams(needs_layout_passes=False),
    )
    def k(xh_h, yh_h, zh_h, feat_h, rv_h, rows_h, cols_h, qx_h, qy_h, qz_h,
          out_h, rows_v, cols_v, qx_v, qy_v, qz_v, idxb, candb, cx, cy, cz,
          seli, selx, sely, selz, idxg2, parb, zfb, featb, obuf, sem,
          fsem0, fsem1):
        wid = lax.axis_index("s") * 2 + lax.axis_index("c")
        base = wid * QPT
        pltpu.sync_copy(rows_h.at[pl.ds(base, QPT)], rows_v)
        pltpu.sync_copy(cols_h.at[pl.ds(base, QPT)], cols_v)
        pltpu.sync_copy(qx_h.at[pl.ds(base, QPT)], qx_v)
        pltpu.sync_copy(qy_h.at[pl.ds(base, QPT)], qy_v)
        pltpu.sync_copy(qz_h.at[pl.ds(base, QPT)], qz_v)

        iota = lax.iota(jnp.int32, 16)
        iota16x = iota * _NSAMPLE

        def window_rel(j):
            kv = iota + 16 * j
            hq = lax.shift_right_logical(kv * 94, 10)
            wq = kv - hq * _W_RANGE
            return hq - 2, wq - 5, kv

        def sel_query(qq, r, c, qxs, qys, qzs):
            cnt = jnp.int32(0)
            for j in range(_KPAD // 16):
                p = qq * _KPAD + j * 16
                rw, off = p // 128, p % 128
                cand = candb[rw, pl.ds(off, 16)]
                xs = cx[rw, pl.ds(off, 16)]
                ys = cy[rw, pl.ds(off, 16)]
                zs = cz[rw, pl.ds(off, 16)]
                hr, wr, kv = window_rel(j)
                rp = r + hr
                cp = c + wr
                ok = (rp >= 0) & (rp < _RH) & (cp >= 0) & (cp < _RW)
                if j == 3:
                    ok = ok & (kv < _H_RANGE * _W_RANGE)
                dx = xs - qxs
                dy = ys - qys
                dz = zs - qzs
                d2 = dx * dx + dy * dy + dz * dz
                inb = ok & (cand >= 0) & (d2 <= _RADIUS2)
                bi = jnp.where(inb, 1, 0)
                pos = plsc.cumsum(bi) - 1 + cnt
                plsc.store_scatter(seli, [pos], cand, mask=inb)
                plsc.store_scatter(selx, [pos], xs, mask=inb)
                plsc.store_scatter(sely, [pos], ys, mask=inb)
                plsc.store_scatter(selz, [pos], zs, mask=inb)
                cnt = cnt + jnp.sum(bi)

            nonempty = cnt > 0
            pos16 = jnp.where(iota < cnt, iota, 0)
            gi = plsc.load_gather(seli, [pos16])
            gx = plsc.load_gather(selx, [pos16])
            gy = plsc.load_gather(sely, [pos16])
            gz = plsc.load_gather(selz, [pos16])
            gi = jnp.where(nonempty, gi, 0)
            idxg2[qq // 4, pl.ds((qq % 4) * 16, 16)] = (
                lax.shift_right_logical(gi, 1))
            parb[qq, pl.ds(0, 16)] = gi & 1
            zfb[qq, pl.ds(0, 16)] = jnp.where(
                nonempty, jnp.float32(1.0), jnp.float32(0.0)) + (iota * 0.0)
            ob = qq * ROW
            obuf[pl.ds(ob, 16)] = jnp.where(nonempty, gx - qxs, 0.0)
            obuf[pl.ds(ob + 16, 16)] = jnp.where(nonempty, gy - qys, 0.0)
            obuf[pl.ds(ob + 32, 16)] = jnp.where(nonempty, gz - qzs, 0.0)

        def transpose_query(qq, buf):
            parv = parb[qq, pl.ds(0, 16)]
            zfv = zfb[qq, pl.ds(0, 16)]
            for s in range(_NSAMPLE):
                off = pl.multiple_of(parv[s] * C, C)
                frow = (qq % 4) * 16 + s
                for cb in range(C // 16):
                    vec = featb[buf, frow, pl.ds(off + cb * 16, 16)]
                    posv = iota16x + (qq * ROW + 48 + cb * 256 + s)
                    plsc.store_scatter(obuf, [posv], vec * zfv)

        def body(g, carry):
            start = pl.multiple_of(g * G, 16)
            rows16 = rows_v[pl.ds(start, 16)]
            cols16 = cols_v[pl.ds(start, 16)]
            qx16 = qx_v[pl.ds(start, 16)]
            qy16 = qy_v[pl.ds(start, 16)]
            qz16 = qz_v[pl.ds(start, 16)]

            for qq in range(G):
                r = rows16[qq]
                c = cols16[qq]
                for j in range(_KPAD // 16):
                    hr, wr, _ = window_rel(j)
                    rp = r + hr
                    cp = c + wr
                    rc = jnp.clip(rp, 0, _RH - 1)
                    cc = jnp.clip(cp, 0, _RW - 1)
                    p = qq * _KPAD + j * 16
                    idxb[p // 128, pl.ds(p % 128, 16)] = rc * _RW + cc
            cps = [pltpu.async_copy(rv_h.at[idxb.at[rw]], candb.at[rw], sem)
                   for rw in range(8)]
            for cp in cps:
                cp.wait()
            cps = []
            for rw in range(8):
                cps.append(pltpu.async_copy(
                    xh_h.at[candb.at[rw]], cx.at[rw], sem))
                cps.append(pltpu.async_copy(
                    yh_h.at[candb.at[rw]], cy.at[rw], sem))
                cps.append(pltpu.async_copy(
                    zh_h.at[candb.at[rw]], cz.at[rw], sem))
            for cp in cps:
                cp.wait()

            for qq in range(G):
                sel_query(qq, rows16[qq], cols16[qq],
                          qx16[qq], qy16[qq], qz16[qq])

            fsems = [fsem0, fsem1]
            cp0 = pltpu.async_copy(
                feat_h.at[idxg2.at[0]], featb.at[0], fsems[0])
            for h in range(4):
                nxt = None
                if h < 3:
                    nxt = pltpu.async_copy(
                        feat_h.at[idxg2.at[h + 1]], featb.at[(h + 1) % 2],
                        fsems[(h + 1) % 2])
                cp0.wait()
                for qq in range(h * 4, h * 4 + 4):
                    transpose_query(qq, h % 2)
                cp0 = nxt

            pltpu.sync_copy(
                obuf, out_h.at[pl.ds((base + start) * ROW, G * ROW)])
            return carry

        lax.fori_loop(0, NG, body, jnp.int32(0))

    return k(xh, yh, zh, features, rv_flat, rows, cols, qx, qy, qz)


def kernel(xyz, features, query_rv_xyz, query_rv_coords, rv_map):
    M = query_rv_xyz.shape[0]
    C = features.shape[1]
    features = features.reshape(features.shape[0] // 2, 2 * C)
    rv_flat = rv_map.reshape(-1)
    rows = query_rv_coords[:, 1].astype(jnp.int32)
    cols = query_rv_coords[:, 2].astype(jnp.int32)
    xh = xyz[:, 0]
    yh = xyz[:, 1]
    zh = xyz[:, 2]
    qx = query_rv_xyz[:, 0]
    qy = query_rv_xyz[:, 1]
    qz = query_rv_xyz[:, 2]
    out = _sc_kernel(xh, yh, zh, features, rv_flat, rows, cols, qx, qy, qz)
    return out.reshape(M, 3 + C, _NSAMPLE)
